# baseline (speedup 1.0000x reference)
import functools

import jax
import jax.numpy as jnp
from jax import lax
from jax.experimental import pallas as pl
from jax.experimental.pallas import tpu as pltpu
from jax.experimental.pallas import tpu_sc as plsc

B = 16384
D = 16
NC, NS, L = 2, 16, 16
NW = NC * NS
B_PER_W = B // NW
IDX_CHUNK = 128
N_CHUNKS = B_PER_W // IDX_CHUNK

GUMBEL_BETA = 0.01
INV_GB = 100.0
EG2 = 2.0 * 0.5772156649015329 * GUMBEL_BETA
TINY = 1.1754943508222875e-38
LN2 = 0.6931471805599453
SQRT2 = 1.4142135623730951


def _log_mantissa(m):
    t = (m - 1.0) / (m + 1.0)
    t2 = t * t
    return 2.0 * t * (1.0 + t2 * (1.0 / 3.0 + t2 * (0.2 + t2 * (1.0 / 7.0))))


def _fast_log(x):
    bits = plsc.bitcast(x, jnp.int32)
    e = (lax.shift_right_logical(bits, 23) & 0xFF) - 127
    m = plsc.bitcast((bits & 0x007FFFFF) | 0x3F800000, jnp.float32)
    big = m > SQRT2
    m = jnp.where(big, 0.5 * m, m)
    ef = e.astype(jnp.float32) + jnp.where(big, 1.0, 0.0)
    return ef * LN2 + _log_mantissa(m)


def _log1p_exp(a):
    z = jnp.exp(a)
    s = 1.0 + z
    r = z - (s - 1.0)
    big = s > SQRT2
    m = jnp.where(big, 0.5 * s, s)
    ef = jnp.where(big, LN2, 0.0)
    return ef + _log_mantissa(m) + r


def _log_softplus(x):
    sp = jnp.maximum(x, 0.0) + _log1p_exp(-jnp.abs(x))
    return _fast_log(jnp.maximum(sp, TINY))


def _sc_body(h_hbm, r_hbm, t_hbm, mine_hbm, dele_hbm, rth_hbm, rsh_hbm,
             rtt_hbm, rst_hbm, out_hbm,
             hidx, ridx, tidx, mn_h, dl_h, mn_t, dl_t, tr_h, sc_h, tr_t, sc_t,
             out_v, sem):
    wid = lax.axis_index("s") * NC + lax.axis_index("c")
    base = wid * B_PER_W

    for k in range(N_CHUNKS):
        off = base + k * IDX_CHUNK
        pltpu.sync_copy(h_hbm.at[pl.ds(off, IDX_CHUNK)], hidx.at[k])
        pltpu.sync_copy(r_hbm.at[pl.ds(off, IDX_CHUNK)], ridx.at[k])
        pltpu.sync_copy(t_hbm.at[pl.ds(off, IDX_CHUNK)], tidx.at[k])

    copies = []
    for k in range(N_CHUNKS):
        rows = pl.ds(k * IDX_CHUNK, IDX_CHUNK)
        for table, idx, dst in (
            (mine_hbm, hidx, mn_h), (dele_hbm, hidx, dl_h),
            (mine_hbm, tidx, mn_t), (dele_hbm, tidx, dl_t),
            (rth_hbm, ridx, tr_h), (rsh_hbm, ridx, sc_h),
            (rtt_hbm, ridx, tr_t), (rst_hbm, ridx, sc_t),
        ):
            copies.append(
                pltpu.async_copy(table.at[idx.at[k]], dst.at[rows, :], sem))
    for c in copies:
        c.wait()

    lane = lax.iota(jnp.int32, L)

    def group(g, _):
        rows = g * L + lane
        acc = jnp.zeros((L,), jnp.float32)
        for d in range(D):
            dv = jnp.full((L,), d, jnp.int32)
            mh = plsc.load_gather(mn_h, [rows, dv])
            dh = plsc.load_gather(dl_h, [rows, dv])
            mt = plsc.load_gather(mn_t, [rows, dv])
            dt = plsc.load_gather(dl_t, [rows, dv])
            th = plsc.load_gather(tr_h, [rows, dv])
            sh = plsc.load_gather(sc_h, [rows, dv])
            tt = plsc.load_gather(tr_t, [rows, dv])
            st = plsc.load_gather(sc_t, [rows, dv])

            h_mn = mh + th
            h_mx = h_mn + jnp.exp(dh) * jnp.maximum(sh, 0.0)
            t_mn = mt + tt
            t_mx = t_mn + jnp.exp(dt) * jnp.maximum(st, 0.0)

            i_mn = jnp.maximum(h_mn, t_mn) + GUMBEL_BETA * _log1p_exp(
                -jnp.abs(h_mn - t_mn) * INV_GB)
            i_mx = jnp.minimum(h_mx, t_mx) - GUMBEL_BETA * _log1p_exp(
                -jnp.abs(h_mx - t_mx) * INV_GB)

            acc += _log_softplus((i_mx - i_mn) - EG2)
            acc -= _log_softplus((t_mx - t_mn) - EG2)
        out_v[pl.ds(g * L, L)] = acc
        return ()

    lax.fori_loop(0, B_PER_W // L, group, ())
    pltpu.sync_copy(out_v, out_hbm.at[pl.ds(base, B_PER_W)])


@jax.jit
def _bi_gumbel_box_sc(h_ids, r_ids, t_ids, min_embedding, delta_embedding,
                      rel_trans_for_head, rel_scale_for_head,
                      rel_trans_for_tail, rel_scale_for_tail):
    mesh = plsc.VectorSubcoreMesh(core_axis_name="c", subcore_axis_name="s",
                                  num_cores=NC, num_subcores=NS)
    run = pl.kernel(
        _sc_body,
        out_type=jax.ShapeDtypeStruct((B,), jnp.float32),
        mesh=mesh,
        compiler_params=pltpu.CompilerParams(needs_layout_passes=False,
                                             use_tc_tiling_on_sc=False),
        scratch_types=[
            pltpu.VMEM((N_CHUNKS, IDX_CHUNK), jnp.int32),
            pltpu.VMEM((N_CHUNKS, IDX_CHUNK), jnp.int32),
            pltpu.VMEM((N_CHUNKS, IDX_CHUNK), jnp.int32),
        ] + [pltpu.VMEM((B_PER_W, D), jnp.float32)] * 8 + [
            pltpu.VMEM((B_PER_W,), jnp.float32),
            pltpu.SemaphoreType.DMA,
        ],
    )
    return run(h_ids, r_ids, t_ids, min_embedding, delta_embedding,
               rel_trans_for_head, rel_scale_for_head,
               rel_trans_for_tail, rel_scale_for_tail)


def kernel(ids, probs, min_embedding, delta_embedding, rel_trans_for_head,
           rel_scale_for_head, rel_trans_for_tail, rel_scale_for_tail):
    h_ids = ids[:, 0].astype(jnp.int32)
    r_ids = ids[:, 1].astype(jnp.int32)
    t_ids = ids[:, 2].astype(jnp.int32)
    log_prob = _bi_gumbel_box_sc(
        h_ids, r_ids, t_ids, min_embedding, delta_embedding,
        rel_trans_for_head, rel_scale_for_head,
        rel_trans_for_tail, rel_scale_for_tail)
    return (log_prob, probs)


# speedup vs baseline: 1.0675x; 1.0675x over previous
import jax
import jax.numpy as jnp
from jax import lax
from jax.experimental import pallas as pl
from jax.experimental.pallas import tpu as pltpu
from jax.experimental.pallas import tpu_sc as plsc

B = 16384
D = 16
NC, NS, L = 2, 16, 16
NW = NC * NS
B_PER_W = B // NW
CHUNK = 32
N_CHUNKS = B_PER_W // CHUNK
GROUPS_PER_CHUNK = CHUNK // L

GUMBEL_BETA = 0.01
INV_GB = 100.0
EG2 = 2.0 * 0.5772156649015329 * GUMBEL_BETA
TINY = 1.1754943508222875e-38
LN2 = 0.6931471805599453
SQRT_HALF_BITS = 0x3F3504F3

_LOG_C = (-1.4097389054723575e-11, 0.9999998807907104, -0.49999991059303284,
          0.3333507776260376, -0.2500225603580475, 0.19936639070510864,
          -0.16551056504249573, 0.15102536976337433, -0.14478063583374023,
          0.08491219580173492)


def _log_poly(u):
    acc = jnp.full_like(u, _LOG_C[-1])
    for c in reversed(_LOG_C[:-1]):
        acc = acc * u + c
    return acc


def _fast_log(x):
    bits = plsc.bitcast(x, jnp.int32)
    k = lax.shift_right_arithmetic(bits - SQRT_HALF_BITS, 23)
    m = plsc.bitcast(bits - lax.shift_left(k, 23), jnp.float32)
    return k.astype(jnp.float32) * LN2 + _log_poly(m - 1.0)


def _log1p_exp(a):
    z = jnp.exp(a)
    s = 1.0 + z
    r = z - (s - 1.0)
    return _fast_log(s) + r


def _log_softplus(x):
    sp = jnp.maximum(x, 0.0) + _log1p_exp(-jnp.abs(x))
    return _fast_log(jnp.maximum(sp, TINY))


def _sc_body(h_hbm, r_hbm, t_hbm, mine_hbm, dele_hbm, rth_hbm, rsh_hbm,
             rtt_hbm, rst_hbm, out_hbm,
             hids, rids, tids, hi_h, hi_r, hi_t,
             b_mnh, b_dlh, b_mnt, b_dlt, b_trh, b_sch, b_trt, b_sct,
             out_v, sem):
    wid = lax.axis_index("s") * NC + lax.axis_index("c")
    base = wid * B_PER_W

    pltpu.sync_copy(h_hbm.at[pl.ds(base, B_PER_W)], hids)
    pltpu.sync_copy(r_hbm.at[pl.ds(base, B_PER_W)], rids)
    pltpu.sync_copy(t_hbm.at[pl.ds(base, B_PER_W)], tids)

    for c in range(B_PER_W // L):
        sl = pl.ds((c % GROUPS_PER_CHUNK) * L, L)
        hi_h[c // GROUPS_PER_CHUNK, sl] = lax.shift_right_logical(
            hids[pl.ds(c * L, L)], 3)
        hi_r[c // GROUPS_PER_CHUNK, sl] = lax.shift_right_logical(
            rids[pl.ds(c * L, L)], 3)
        hi_t[c // GROUPS_PER_CHUNK, sl] = lax.shift_right_logical(
            tids[pl.ds(c * L, L)], 3)

    lane = lax.iota(jnp.int32, L)

    def chunk(k, _):
        copies = [
            pltpu.async_copy(mine_hbm.at[hi_h.at[k]], b_mnh, sem),
            pltpu.async_copy(dele_hbm.at[hi_h.at[k]], b_dlh, sem),
            pltpu.async_copy(mine_hbm.at[hi_t.at[k]], b_mnt, sem),
            pltpu.async_copy(dele_hbm.at[hi_t.at[k]], b_dlt, sem),
            pltpu.async_copy(rth_hbm.at[hi_r.at[k]], b_trh, sem),
            pltpu.async_copy(rsh_hbm.at[hi_r.at[k]], b_sch, sem),
            pltpu.async_copy(rtt_hbm.at[hi_r.at[k]], b_trt, sem),
            pltpu.async_copy(rst_hbm.at[hi_r.at[k]], b_sct, sem),
        ]
        for cp in copies:
            cp.wait()

        def group(gg, _):
            off = k * CHUNK + gg * L
            rows = gg * L + lane
            col_h = (hids[pl.ds(off, L)] & 7) * 16
            col_r = (rids[pl.ds(off, L)] & 7) * 16
            col_t = (tids[pl.ds(off, L)] & 7) * 16

            def dim(d, acc):
                ch = col_h + d
                cr = col_r + d
                ct = col_t + d
                mh = plsc.load_gather(b_mnh, [rows, ch])
                dh = plsc.load_gather(b_dlh, [rows, ch])
                mt = plsc.load_gather(b_mnt, [rows, ct])
                dt = plsc.load_gather(b_dlt, [rows, ct])
                th = plsc.load_gather(b_trh, [rows, cr])
                sh = plsc.load_gather(b_sch, [rows, cr])
                tt = plsc.load_gather(b_trt, [rows, cr])
                st = plsc.load_gather(b_sct, [rows, cr])

                h_mn = mh + th
                h_mx = h_mn + jnp.exp(dh) * jnp.maximum(sh, 0.0)
                t_mn = mt + tt
                t_mx = t_mn + jnp.exp(dt) * jnp.maximum(st, 0.0)

                i_mn = jnp.maximum(h_mn, t_mn) + GUMBEL_BETA * _log1p_exp(
                    -jnp.abs(h_mn - t_mn) * INV_GB)
                i_mx = jnp.minimum(h_mx, t_mx) - GUMBEL_BETA * _log1p_exp(
                    -jnp.abs(h_mx - t_mx) * INV_GB)

                acc += _log_softplus((i_mx - i_mn) - EG2)
                acc -= _log_softplus((t_mx - t_mn) - EG2)
                return acc

            out_v[pl.ds(off, L)] = lax.fori_loop(
                0, D, dim, jnp.zeros((L,), jnp.float32))
            return ()

        lax.fori_loop(0, GROUPS_PER_CHUNK, group, ())
        return ()

    lax.fori_loop(0, N_CHUNKS, chunk, ())
    pltpu.sync_copy(out_v, out_hbm.at[pl.ds(base, B_PER_W)])


@jax.jit
def _bi_gumbel_box_sc(h_ids, r_ids, t_ids, mine_blk, dele_blk,
                      rth_blk, rsh_blk, rtt_blk, rst_blk):
    mesh = plsc.VectorSubcoreMesh(core_axis_name="c", subcore_axis_name="s",
                                  num_cores=NC, num_subcores=NS)
    run = pl.kernel(
        _sc_body,
        out_type=jax.ShapeDtypeStruct((B,), jnp.float32),
        mesh=mesh,
        compiler_params=pltpu.CompilerParams(needs_layout_passes=False),
        scratch_types=[
            pltpu.VMEM((B_PER_W,), jnp.int32),
            pltpu.VMEM((B_PER_W,), jnp.int32),
            pltpu.VMEM((B_PER_W,), jnp.int32),
            pltpu.VMEM((N_CHUNKS, CHUNK), jnp.int32),
            pltpu.VMEM((N_CHUNKS, CHUNK), jnp.int32),
            pltpu.VMEM((N_CHUNKS, CHUNK), jnp.int32),
        ] + [pltpu.VMEM((CHUNK, 128), jnp.float32)] * 8 + [
            pltpu.VMEM((B_PER_W,), jnp.float32),
            pltpu.SemaphoreType.DMA,
        ],
    )
    return run(h_ids, r_ids, t_ids, mine_blk, dele_blk,
               rth_blk, rsh_blk, rtt_blk, rst_blk)


def kernel(ids, probs, min_embedding, delta_embedding, rel_trans_for_head,
           rel_scale_for_head, rel_trans_for_tail, rel_scale_for_tail):
    h_ids = ids[:, 0].astype(jnp.int32)
    r_ids = ids[:, 1].astype(jnp.int32)
    t_ids = ids[:, 2].astype(jnp.int32)
    log_prob = _bi_gumbel_box_sc(
        h_ids, r_ids, t_ids,
        min_embedding.reshape(-1, 128), delta_embedding.reshape(-1, 128),
        rel_trans_for_head.reshape(-1, 128),
        rel_scale_for_head.reshape(-1, 128),
        rel_trans_for_tail.reshape(-1, 128),
        rel_scale_for_tail.reshape(-1, 128))
    return (log_prob, probs)
